# baseline (speedup 1.0000x reference)
import jax
import jax.numpy as jnp
from jax import lax
from jax.experimental import pallas as pl
from jax.experimental.pallas import tpu as pltpu
from jax.experimental.pallas import tpu_sc as plsc

FD = 32
NPTS = 262144
NWORKERS = 32
PPW = NPTS // NWORKERS
CHUNK = 128
NCHUNK = PPW // CHUNK
NT = 3 * NCHUNK
L = 16
PRS = ((0, 1), (0, 2), (1, 2))


def _sc_body(xT, table, out, xv, ibuf, wbuf, vbuf, acc, xsem, gsem, osem):
    cid = lax.axis_index("c")
    sid = lax.axis_index("s")
    wid = sid * 2 + cid

    def decode(t):
        s = lax.shift_right_logical(t, 6)
        g = lax.bitwise_and(t, NCHUNK - 1)
        return s, g

    def pbase(t):
        _, g = decode(t)
        return wid * PPW + g * CHUNK

    def fire_x(t, buf):
        return pltpu.async_copy(
            xT.at[:, pl.ds(pbase(t), CHUNK)], xv.at[buf], xsem.at[buf])

    def stage_a(t, buf):
        s, _ = decode(t)
        r = lax.shift_left(128, s)
        h = (r - 1).astype(jnp.float32) * 0.5
        rm1 = r - 1
        q4 = lax.shift_left(1, 2 * s)
        rsq = 16384 * q4
        sbase = 16384 * (q4 - 1)

        def slice_body(u, _):
            o = u * L
            xs = [xv[buf, i, pl.ds(o, L)] for i in range(3)]
            for k, (a, b) in enumerate(PRS):
                pb = sbase + k * rsq
                gx = xs[a] * h + h
                gy = xs[b] * h + h
                x0 = jnp.minimum(gx.astype(jnp.int32), rm1)
                y0 = jnp.minimum(gy.astype(jnp.int32), rm1)
                wx1 = gx - x0.astype(jnp.float32)
                wy1 = gy - y0.astype(jnp.float32)
                wx0 = 1.0 - wx1
                wy0 = 1.0 - wy1
                x1 = jnp.minimum(x0 + 1, rm1)
                y1 = jnp.minimum(y0 + 1, rm1)
                yb0 = y0 * r + pb
                yb1 = y1 * r + pb
                ibuf[buf, 4 * k + 0, pl.ds(o, L)] = yb0 + x0
                ibuf[buf, 4 * k + 1, pl.ds(o, L)] = yb0 + x1
                ibuf[buf, 4 * k + 2, pl.ds(o, L)] = yb1 + x0
                ibuf[buf, 4 * k + 3, pl.ds(o, L)] = yb1 + x1
                wbuf[buf, 4 * k + 0, pl.ds(o, L)] = wy0 * wx0
                wbuf[buf, 4 * k + 1, pl.ds(o, L)] = wy0 * wx1
                wbuf[buf, 4 * k + 2, pl.ds(o, L)] = wy1 * wx0
                wbuf[buf, 4 * k + 3, pl.ds(o, L)] = wy1 * wx1
            return 0

        lax.fori_loop(0, CHUNK // L, slice_body, 0)

    def fire_gathers(buf):
        for q in range(12):
            pltpu.async_copy(table.at[ibuf.at[buf, q]], vbuf.at[buf, q],
                             gsem.at[buf])

    def wait_gathers(buf):
        for q in range(12):
            pltpu.make_async_copy(table.at[ibuf.at[buf, q]],
                                  vbuf.at[buf, q], gsem.at[buf]).wait()

    def out_slice(t):
        s, _ = decode(t)
        return out.at[pl.ds(pbase(t), CHUNK), pl.ds(FD * s, FD)]

    def combine(buf):
        mhi = jnp.int32(-65536)

        def group_body(pg, _):
            po = pg * L
            wv = [wbuf[buf, q, pl.ds(po, L)] for q in range(12)]
            for u in range(L):
                p = po + u
                flo = fhi = None
                for k in range(3):
                    vlo = vhi = None
                    for c in range(4):
                        row = vbuf[buf, 4 * k + c, p, :]
                        lo = lax.bitcast_convert_type(
                            lax.shift_left(row, 16), jnp.float32)
                        hi = lax.bitcast_convert_type(
                            lax.bitwise_and(row, mhi), jnp.float32)
                        w = wv[4 * k + c][u]
                        if vlo is None:
                            vlo, vhi = lo * w, hi * w
                        else:
                            vlo, vhi = vlo + lo * w, vhi + hi * w
                    if flo is None:
                        flo, fhi = vlo, vhi
                    else:
                        flo, fhi = flo * vlo, fhi * vhi
                acc[buf, p, pl.ds(0, L)] = flo
                acc[buf, p, pl.ds(L, L)] = fhi
            return 0

        lax.fori_loop(0, CHUNK // L, group_body, 0)

    fire_x(0, 0)
    fire_x(1, 1)
    pltpu.make_async_copy(xT.at[:, pl.ds(pbase(0), CHUNK)], xv.at[0],
                          xsem.at[0]).wait()
    stage_a(0, 0)
    fire_gathers(0)

    def body(t, _):
        par = lax.bitwise_and(t, 1)
        nxt = 1 - par

        wait_gathers(par)

        @pl.when(t < NT - 1)
        def _():
            pltpu.make_async_copy(xT.at[:, pl.ds(pbase(t + 1), CHUNK)],
                                  xv.at[nxt], xsem.at[nxt]).wait()
            stage_a(t + 1, nxt)
            fire_gathers(nxt)

        @pl.when(t < NT - 2)
        def _():
            fire_x(t + 2, par)

        @pl.when(t >= 2)
        def _():
            pltpu.make_async_copy(acc.at[par], out_slice(t - 2),
                                  osem.at[par]).wait()

        combine(par)
        pltpu.async_copy(acc.at[par], out_slice(t), osem.at[par])
        return 0

    lax.fori_loop(0, NT, body, 0)

    pltpu.make_async_copy(acc.at[0], out_slice(NT - 2), osem.at[0]).wait()
    pltpu.make_async_copy(acc.at[1], out_slice(NT - 1), osem.at[1]).wait()


_PERM = tuple(
    int(v) for pair in zip(range(16), range(16, 32)) for v in pair)


def kernel(x, p00, p01, p02, p10, p11, p12, p20, p21, p22):
    xT = x.T
    perm = jnp.array(_PERM, dtype=jnp.int32)
    table = jnp.concatenate(
        [jnp.transpose(p, (1, 2, 0)).reshape(-1, FD)
         for p in (p00, p01, p02, p10, p11, p12, p20, p21, p22)], axis=0)
    table = jax.lax.bitcast_convert_type(
        table[:, perm].astype(jnp.bfloat16).reshape(-1, FD // 2, 2),
        jnp.int32)
    mesh = plsc.VectorSubcoreMesh(core_axis_name="c", subcore_axis_name="s")
    f = pl.kernel(
        _sc_body,
        out_type=jax.ShapeDtypeStruct((NPTS, 3 * FD), jnp.float32),
        mesh=mesh,
        compiler_params=pltpu.CompilerParams(use_tc_tiling_on_sc=False),
        scratch_types=[
            pltpu.VMEM((2, 3, CHUNK), jnp.float32),
            pltpu.VMEM((2, 12, CHUNK), jnp.int32),
            pltpu.VMEM((2, 12, CHUNK), jnp.float32),
            pltpu.VMEM((2, 12, CHUNK, FD // 2), jnp.int32),
            pltpu.VMEM((2, CHUNK, FD), jnp.float32),
            pltpu.SemaphoreType.DMA((2,)),
            pltpu.SemaphoreType.DMA((2,)),
            pltpu.SemaphoreType.DMA((2,)),
        ],
    )
    return f(xT, table)


# speedup vs baseline: 1.4224x; 1.4224x over previous
import jax
import jax.numpy as jnp
from jax import lax
from jax.experimental import pallas as pl
from jax.experimental.pallas import tpu as pltpu
from jax.experimental.pallas import tpu_sc as plsc

FD = 32
NPTS = 262144
NWORKERS = 32
PPW = NPTS // NWORKERS
CHUNK = 128
NCHUNK = PPW // CHUNK
NT = 3 * NCHUNK
L = 16
PRS = ((0, 1), (0, 2), (1, 2))


def _sc_body(xT, table, out, xv, ibuf, wbuf, vbuf, acc, xsem, gsem, osem):
    cid = lax.axis_index("c")
    sid = lax.axis_index("s")
    wid = sid * 2 + cid

    def decode(t):
        s = lax.shift_right_logical(t, 6)
        g = lax.bitwise_and(t, NCHUNK - 1)
        return s, g

    def pbase(t):
        _, g = decode(t)
        return wid * PPW + g * CHUNK

    def fire_x(t, buf):
        return pltpu.async_copy(
            xT.at[:, pl.ds(pbase(t), CHUNK)], xv.at[buf], xsem.at[buf])

    def stage_a(t, buf):
        s, _ = decode(t)
        r = lax.shift_left(128, s)
        h = (r - 1).astype(jnp.float32) * 0.5
        rm1 = r - 1
        q4 = lax.shift_left(1, 2 * s)
        rsq = 16384 * q4
        sbase = 16384 * (q4 - 1)

        def slice_body(u, _):
            o = u * L
            xs = [xv[buf, i, pl.ds(o, L)] for i in range(3)]
            for k, (a, b) in enumerate(PRS):
                pb = sbase + k * rsq
                gx = xs[a] * h + h
                gy = xs[b] * h + h
                x0 = jnp.minimum(gx.astype(jnp.int32), rm1)
                y0 = jnp.minimum(gy.astype(jnp.int32), rm1)
                wx1 = gx - x0.astype(jnp.float32)
                wy1 = gy - y0.astype(jnp.float32)
                wx0 = 1.0 - wx1
                wy0 = 1.0 - wy1
                x1 = jnp.minimum(x0 + 1, rm1)
                y1 = jnp.minimum(y0 + 1, rm1)
                yb0 = y0 * r + pb
                yb1 = y1 * r + pb
                ibuf[buf, 4 * k + 0, pl.ds(o, L)] = yb0 + x0
                ibuf[buf, 4 * k + 1, pl.ds(o, L)] = yb0 + x1
                ibuf[buf, 4 * k + 2, pl.ds(o, L)] = yb1 + x0
                ibuf[buf, 4 * k + 3, pl.ds(o, L)] = yb1 + x1
                wbuf[buf, 4 * k + 0, pl.ds(o, L)] = wy0 * wx0
                wbuf[buf, 4 * k + 1, pl.ds(o, L)] = wy0 * wx1
                wbuf[buf, 4 * k + 2, pl.ds(o, L)] = wy1 * wx0
                wbuf[buf, 4 * k + 3, pl.ds(o, L)] = wy1 * wx1
            return 0

        lax.fori_loop(0, CHUNK // L, slice_body, 0)

    def fire_gathers(buf):
        for q in range(12):
            pltpu.async_copy(table.at[ibuf.at[buf, q]], vbuf.at[buf, q],
                             gsem.at[buf])

    def wait_gathers(buf):
        for q in range(12):
            pltpu.make_async_copy(table.at[ibuf.at[buf, q]],
                                  vbuf.at[buf, q], gsem.at[buf]).wait()

    def out_slice(t):
        s, _ = decode(t)
        return out.at[pl.ds(pbase(t), CHUNK), pl.ds(FD * s, FD)]

    def combine(buf):
        mhi = jnp.int32(-65536)

        def group_body(pg, _):
            po = pg * L
            wv = [wbuf[buf, q, pl.ds(po, L)] for q in range(12)]
            for u in range(L):
                p = po + u
                flo = fhi = None
                for k in range(3):
                    vlo = vhi = None
                    for c in range(4):
                        row = vbuf[buf, 4 * k + c, p, :]
                        lo = lax.bitcast_convert_type(
                            lax.shift_left(row, 16), jnp.float32)
                        hi = lax.bitcast_convert_type(
                            lax.bitwise_and(row, mhi), jnp.float32)
                        w = wv[4 * k + c][u]
                        if vlo is None:
                            vlo, vhi = lo * w, hi * w
                        else:
                            vlo, vhi = vlo + lo * w, vhi + hi * w
                    if flo is None:
                        flo, fhi = vlo, vhi
                    else:
                        flo, fhi = flo * vlo, fhi * vhi
                acc[buf, p, pl.ds(0, L)] = flo
                acc[buf, p, pl.ds(L, L)] = fhi
            return 0

        lax.fori_loop(0, CHUNK // L, group_body, 0)

    fire_x(0, 0)
    fire_x(1, 1)
    pltpu.make_async_copy(xT.at[:, pl.ds(pbase(0), CHUNK)], xv.at[0],
                          xsem.at[0]).wait()
    stage_a(0, 0)
    fire_gathers(0)

    def body(t, _):
        par = lax.bitwise_and(t, 1)
        nxt = 1 - par

        wait_gathers(par)

        @pl.when(t < NT - 1)
        def _():
            pltpu.make_async_copy(xT.at[:, pl.ds(pbase(t + 1), CHUNK)],
                                  xv.at[nxt], xsem.at[nxt]).wait()
            stage_a(t + 1, nxt)
            fire_gathers(nxt)

        @pl.when(t < NT - 2)
        def _():
            fire_x(t + 2, par)

        @pl.when(t >= 2)
        def _():
            pltpu.make_async_copy(acc.at[par], out_slice(t - 2),
                                  osem.at[par]).wait()

        combine(par)
        pltpu.async_copy(acc.at[par], out_slice(t), osem.at[par])
        return 0

    lax.fori_loop(0, NT, body, 0)

    pltpu.make_async_copy(acc.at[0], out_slice(NT - 2), osem.at[0]).wait()
    pltpu.make_async_copy(acc.at[1], out_slice(NT - 1), osem.at[1]).wait()


def kernel(x, p00, p01, p02, p10, p11, p12, p20, p21, p22):
    xT = x.T
    table = jnp.concatenate(
        [jnp.transpose(p.reshape(2, FD // 2, -1), (2, 1, 0))
         for p in (p00, p01, p02, p10, p11, p12, p20, p21, p22)], axis=0)
    table = jax.lax.bitcast_convert_type(
        table.astype(jnp.bfloat16), jnp.int32)
    mesh = plsc.VectorSubcoreMesh(core_axis_name="c", subcore_axis_name="s")
    f = pl.kernel(
        _sc_body,
        out_type=jax.ShapeDtypeStruct((NPTS, 3 * FD), jnp.float32),
        mesh=mesh,
        compiler_params=pltpu.CompilerParams(use_tc_tiling_on_sc=False),
        scratch_types=[
            pltpu.VMEM((2, 3, CHUNK), jnp.float32),
            pltpu.VMEM((2, 12, CHUNK), jnp.int32),
            pltpu.VMEM((2, 12, CHUNK), jnp.float32),
            pltpu.VMEM((2, 12, CHUNK, FD // 2), jnp.int32),
            pltpu.VMEM((2, CHUNK, FD), jnp.float32),
            pltpu.SemaphoreType.DMA((2,)),
            pltpu.SemaphoreType.DMA((2,)),
            pltpu.SemaphoreType.DMA((2,)),
        ],
    )
    return f(xT, table)


# speedup vs baseline: 1.5321x; 1.0772x over previous
import jax
import jax.numpy as jnp
from jax import lax
from jax.experimental import pallas as pl
from jax.experimental.pallas import tpu as pltpu
from jax.experimental.pallas import tpu_sc as plsc

FD = 32
NPTS = 262144
NWORKERS = 32
PPW = NPTS // NWORKERS
CHUNK = 128
NCHUNK = PPW // CHUNK
NT = 3 * NCHUNK
L = 16
PRS = ((0, 1), (0, 2), (1, 2))


def _sc_body(xT, table, out, xv, ibuf, wbuf, vbuf, acc, xsem, gsem, osem):
    cid = lax.axis_index("c")
    sid = lax.axis_index("s")
    wid = sid * 2 + cid

    def decode(t):
        s = lax.shift_right_logical(t, 6)
        g = lax.bitwise_and(t, NCHUNK - 1)
        return s, g

    def pbase(t):
        _, g = decode(t)
        return wid * PPW + g * CHUNK

    def fire_x(t, buf):
        pltpu.async_copy(xT.at[:, pl.ds(pbase(t), CHUNK)], xv.at[buf],
                         xsem.at[buf])

    def wait_x(t, buf):
        pltpu.make_async_copy(xT.at[:, pl.ds(pbase(t), CHUNK)], xv.at[buf],
                              xsem.at[buf]).wait()

    def stage_a(t, buf):
        s, _ = decode(t)
        r = lax.shift_left(128, s)
        h = (r - 1).astype(jnp.float32) * 0.5
        rm1 = r - 1
        q4 = lax.shift_left(1, 2 * s)
        rsq = 16384 * q4
        sbase = 16384 * (q4 - 1)

        def slice_body(u, _):
            o = u * L
            xs = [xv[buf, i, pl.ds(o, L)] for i in range(3)]
            for k, (a, b) in enumerate(PRS):
                pb = sbase + k * rsq
                gx = xs[a] * h + h
                gy = xs[b] * h + h
                x0 = jnp.minimum(gx.astype(jnp.int32), rm1)
                y0 = jnp.minimum(gy.astype(jnp.int32), rm1)
                wx1 = gx - x0.astype(jnp.float32)
                wy1 = gy - y0.astype(jnp.float32)
                wx0 = 1.0 - wx1
                wy0 = 1.0 - wy1
                x1 = jnp.minimum(x0 + 1, rm1)
                y1 = jnp.minimum(y0 + 1, rm1)
                yb0 = y0 * r + pb
                yb1 = y1 * r + pb
                ibuf[buf, 4 * k + 0, pl.ds(o, L)] = yb0 + x0
                ibuf[buf, 4 * k + 1, pl.ds(o, L)] = yb0 + x1
                ibuf[buf, 4 * k + 2, pl.ds(o, L)] = yb1 + x0
                ibuf[buf, 4 * k + 3, pl.ds(o, L)] = yb1 + x1
                wbuf[buf, 4 * k + 0, pl.ds(o, L)] = wy0 * wx0
                wbuf[buf, 4 * k + 1, pl.ds(o, L)] = wy0 * wx1
                wbuf[buf, 4 * k + 2, pl.ds(o, L)] = wy1 * wx0
                wbuf[buf, 4 * k + 3, pl.ds(o, L)] = wy1 * wx1
            return 0

        lax.fori_loop(0, CHUNK // L, slice_body, 0)

    def fire_gathers(buf):
        for q in range(12):
            pltpu.async_copy(table.at[ibuf.at[buf, q]], vbuf.at[buf, q],
                             gsem.at[buf])

    def wait_gathers(buf):
        for q in range(12):
            pltpu.make_async_copy(table.at[ibuf.at[buf, q]],
                                  vbuf.at[buf, q], gsem.at[buf]).wait()

    def out_slice(t):
        s, _ = decode(t)
        return out.at[pl.ds(pbase(t), CHUNK), pl.ds(FD * s, FD)]

    def combine(buf):
        mhi = jnp.int32(-65536)

        def group_body(pg, _):
            po = pg * L
            wv = [wbuf[buf, q, pl.ds(po, L)] for q in range(12)]
            for u in range(L):
                p = po + u
                flo = fhi = None
                for k in range(3):
                    vlo = vhi = None
                    for c in range(4):
                        row = vbuf[buf, 4 * k + c, p, :]
                        lo = lax.bitcast_convert_type(
                            lax.shift_left(row, 16), jnp.float32)
                        hi = lax.bitcast_convert_type(
                            lax.bitwise_and(row, mhi), jnp.float32)
                        w = wv[4 * k + c][u]
                        if vlo is None:
                            vlo, vhi = lo * w, hi * w
                        else:
                            vlo, vhi = vlo + lo * w, vhi + hi * w
                    if flo is None:
                        flo, fhi = vlo, vhi
                    else:
                        flo, fhi = flo * vlo, fhi * vhi
                acc[buf, p, pl.ds(0, L)] = flo
                acc[buf, p, pl.ds(L, L)] = fhi
            return 0

        lax.fori_loop(0, CHUNK // L, group_body, 0)

    fire_x(0, 0)
    fire_x(1, 1)
    wait_x(0, 0)
    stage_a(0, 0)
    fire_gathers(0)

    def body(t, _):
        par = lax.bitwise_and(t, 1)
        nxt = 1 - par

        wait_gathers(par)

        @pl.when(t < NT - 1)
        def _():
            wait_x(t + 1, nxt)
            stage_a(t + 1, nxt)
            fire_gathers(nxt)

        @pl.when(t < NT - 2)
        def _():
            fire_x(t + 2, par)

        @pl.when(t >= 2)
        def _():
            pltpu.make_async_copy(acc.at[par], out_slice(t - 2),
                                  osem.at[par]).wait()

        combine(par)
        pltpu.async_copy(acc.at[par], out_slice(t), osem.at[par])
        return 0

    lax.fori_loop(0, NT, body, 0)

    pltpu.make_async_copy(acc.at[0], out_slice(NT - 2), osem.at[0]).wait()
    pltpu.make_async_copy(acc.at[1], out_slice(NT - 1), osem.at[1]).wait()


def kernel(x, p00, p01, p02, p10, p11, p12, p20, p21, p22):
    xT = jax.lax.dot_general(jnp.eye(3, dtype=jnp.float32), x,
                             (((1,), (1,)), ((), ())))
    table = jnp.concatenate(
        [jnp.transpose(p.reshape(2, FD // 2, -1), (2, 1, 0))
         for p in (p00, p01, p02, p10, p11, p12, p20, p21, p22)], axis=0)
    table = jax.lax.bitcast_convert_type(
        table.astype(jnp.bfloat16), jnp.int32)
    mesh = plsc.VectorSubcoreMesh(core_axis_name="c", subcore_axis_name="s")
    f = pl.kernel(
        _sc_body,
        out_type=jax.ShapeDtypeStruct((NPTS, 128), jnp.float32),
        mesh=mesh,
        compiler_params=pltpu.CompilerParams(use_tc_tiling_on_sc=False),
        scratch_types=[
            pltpu.VMEM((2, 3, CHUNK), jnp.float32),
            pltpu.VMEM((2, 12, CHUNK), jnp.int32),
            pltpu.VMEM((2, 12, CHUNK), jnp.float32),
            pltpu.VMEM((2, 12, CHUNK, FD // 2), jnp.int32),
            pltpu.VMEM((2, CHUNK, FD), jnp.float32),
            pltpu.SemaphoreType.DMA((2,)),
            pltpu.SemaphoreType.DMA((2,)),
            pltpu.SemaphoreType.DMA((2,)),
        ],
    )
    return f(xT, table)[:, :3 * FD]


# speedup vs baseline: 1.5340x; 1.0012x over previous
import jax
import jax.numpy as jnp
from jax import lax
from jax.experimental import pallas as pl
from jax.experimental.pallas import tpu as pltpu
from jax.experimental.pallas import tpu_sc as plsc

FD = 32
NPTS = 262144
NWORKERS = 32
PPW = NPTS // NWORKERS
CHUNK = 128
NCHUNK = PPW // CHUNK
NT = 3 * NCHUNK
L = 16
PRS = ((0, 1), (0, 2), (1, 2))


def _sc_body(xT, table, out, xv, ibuf, wbuf, vbuf, acc, xsem, gsem, osem):
    cid = lax.axis_index("c")
    sid = lax.axis_index("s")
    wid = sid * 2 + cid

    def decode(t):
        s = lax.shift_right_logical(t, 6)
        g = lax.bitwise_and(t, NCHUNK - 1)
        return s, g

    def pbase(t):
        _, g = decode(t)
        return wid * PPW + g * CHUNK

    def fire_x(t, buf):
        pltpu.async_copy(xT.at[:, pl.ds(pbase(t), CHUNK)], xv.at[buf],
                         xsem.at[buf])

    def wait_x(t, buf):
        pltpu.make_async_copy(xT.at[:, pl.ds(pbase(t), CHUNK)], xv.at[buf],
                              xsem.at[buf]).wait()

    def stage_a(t, buf):
        s, _ = decode(t)
        r = lax.shift_left(128, s)
        h = (r - 1).astype(jnp.float32) * 0.5
        rm1 = r - 1
        q4 = lax.shift_left(1, 2 * s)
        rsq = 16384 * q4
        sbase = 16384 * (q4 - 1)

        def slice_body(u, _):
            o = u * L
            xs = [xv[buf, i, pl.ds(o, L)] for i in range(3)]
            for k, (a, b) in enumerate(PRS):
                pb = sbase + k * rsq
                gx = xs[a] * h + h
                gy = xs[b] * h + h
                x0 = jnp.minimum(gx.astype(jnp.int32), rm1)
                y0 = jnp.minimum(gy.astype(jnp.int32), rm1)
                wx1 = gx - x0.astype(jnp.float32)
                wy1 = gy - y0.astype(jnp.float32)
                wx0 = 1.0 - wx1
                wy0 = 1.0 - wy1
                x1 = jnp.minimum(x0 + 1, rm1)
                y1 = jnp.minimum(y0 + 1, rm1)
                yb0 = y0 * r + pb
                yb1 = y1 * r + pb
                ibuf[buf, 4 * k + 0, pl.ds(o, L)] = yb0 + x0
                ibuf[buf, 4 * k + 1, pl.ds(o, L)] = yb0 + x1
                ibuf[buf, 4 * k + 2, pl.ds(o, L)] = yb1 + x0
                ibuf[buf, 4 * k + 3, pl.ds(o, L)] = yb1 + x1
                wbuf[buf, 4 * k + 0, pl.ds(o, L)] = wy0 * wx0
                wbuf[buf, 4 * k + 1, pl.ds(o, L)] = wy0 * wx1
                wbuf[buf, 4 * k + 2, pl.ds(o, L)] = wy1 * wx0
                wbuf[buf, 4 * k + 3, pl.ds(o, L)] = wy1 * wx1
            return 0

        lax.fori_loop(0, CHUNK // L, slice_body, 0)

    def fire_gathers(buf):
        for q in range(12):
            pltpu.async_copy(table.at[ibuf.at[buf, q]], vbuf.at[buf, q],
                             gsem.at[buf])

    def wait_gathers(buf):
        for q in range(12):
            pltpu.make_async_copy(table.at[ibuf.at[buf, q]],
                                  vbuf.at[buf, q], gsem.at[buf]).wait()

    def out_slice(t):
        s, _ = decode(t)
        return out.at[pl.ds(pbase(t), CHUNK), pl.ds(FD * s, FD)]

    def combine(buf):
        mhi = jnp.int32(-65536)

        def group_body(pg, _):
            po = pg * L
            wv = [wbuf[buf, q, pl.ds(po, L)] for q in range(12)]
            for u in range(L):
                p = po + u
                flo = fhi = None
                for k in range(3):
                    vlo = vhi = None
                    for c in range(4):
                        row = vbuf[buf, 4 * k + c, p, :]
                        lo = lax.bitcast_convert_type(
                            lax.shift_left(row, 16), jnp.float32)
                        hi = lax.bitcast_convert_type(
                            lax.bitwise_and(row, mhi), jnp.float32)
                        w = wv[4 * k + c][u]
                        if vlo is None:
                            vlo, vhi = lo * w, hi * w
                        else:
                            vlo, vhi = vlo + lo * w, vhi + hi * w
                    if flo is None:
                        flo, fhi = vlo, vhi
                    else:
                        flo, fhi = flo * vlo, fhi * vhi
                acc[buf, p, pl.ds(0, L)] = flo
                acc[buf, p, pl.ds(L, L)] = fhi
            return 0

        lax.fori_loop(0, CHUNK // L, group_body, 0)

    fire_x(0, 0)
    fire_x(1, 1)
    wait_x(0, 0)
    stage_a(0, 0)
    fire_gathers(0)

    def body(t, _):
        par = lax.bitwise_and(t, 1)
        nxt = 1 - par

        wait_gathers(par)

        @pl.when(t < NT - 1)
        def _():
            wait_x(t + 1, nxt)
            stage_a(t + 1, nxt)
            fire_gathers(nxt)

        @pl.when(t < NT - 2)
        def _():
            fire_x(t + 2, par)

        @pl.when(t >= 2)
        def _():
            pltpu.make_async_copy(acc.at[par], out_slice(t - 2),
                                  osem.at[par]).wait()

        combine(par)
        pltpu.async_copy(acc.at[par], out_slice(t), osem.at[par])
        return 0

    lax.fori_loop(0, NT, body, 0)

    pltpu.make_async_copy(acc.at[0], out_slice(NT - 2), osem.at[0]).wait()
    pltpu.make_async_copy(acc.at[1], out_slice(NT - 1), osem.at[1]).wait()


def kernel(x, p00, p01, p02, p10, p11, p12, p20, p21, p22):
    xT = jax.lax.dot_general(jnp.eye(3, dtype=jnp.float32), x,
                             (((1,), (1,)), ((), ())),
                             precision=jax.lax.Precision.HIGHEST)
    table = jnp.concatenate(
        [jnp.transpose(p.reshape(2, FD // 2, -1), (2, 1, 0))
         for p in (p00, p01, p02, p10, p11, p12, p20, p21, p22)], axis=0)
    table = jax.lax.bitcast_convert_type(
        table.astype(jnp.bfloat16), jnp.int32)
    mesh = plsc.VectorSubcoreMesh(core_axis_name="c", subcore_axis_name="s")
    f = pl.kernel(
        _sc_body,
        out_type=jax.ShapeDtypeStruct((NPTS, 128), jnp.float32),
        mesh=mesh,
        compiler_params=pltpu.CompilerParams(use_tc_tiling_on_sc=False),
        scratch_types=[
            pltpu.VMEM((2, 3, CHUNK), jnp.float32),
            pltpu.VMEM((2, 12, CHUNK), jnp.int32),
            pltpu.VMEM((2, 12, CHUNK), jnp.float32),
            pltpu.VMEM((2, 12, CHUNK, FD // 2), jnp.int32),
            pltpu.VMEM((2, CHUNK, FD), jnp.float32),
            pltpu.SemaphoreType.DMA((2,)),
            pltpu.SemaphoreType.DMA((2,)),
            pltpu.SemaphoreType.DMA((2,)),
        ],
    )
    return f(xT, table)[:, :3 * FD]
